# baseline (speedup 1.0000x reference)
import functools

import jax
import numpy as np
import jax.numpy as jnp
from jax import lax
from jax.experimental import pallas as pl
from jax.experimental.pallas import tpu as pltpu
from jax.experimental.pallas import tpu_sc as plsc

N = 10000
E = 160000
D = 256
DE = 16
H = 128
M_BLK = 1000
E_BLK = 2048
CH = 64
N_SUB = 16
EP = 163840
EPT = EP // N_SUB
N_CH = EPT // CH
NP = 10240
RPT = NP // N_SUB


def _make_ew_perm():
    p = []
    for half in range(2):
        base = half * H
        for j in range(H // 32):
            for i in range(16):
                p.append(base + 32 * j + i)
                p.append(base + 32 * j + 16 + i)
    return np.array(p, dtype=np.int32)


_EW_PERM = _make_ew_perm()


def _prep_body(x_ref, wlt_ref, w1at_ref, h_ref, hw_ref):
    h = jnp.maximum(
        jnp.dot(x_ref[...], wlt_ref[...], preferred_element_type=jnp.float32), 0.0)
    h_ref[...] = h
    hw = jnp.dot(h, w1at_ref[...], preferred_element_type=jnp.float32)
    hw_ref[0] = hw[:, :H]
    hw_ref[1] = hw[:, H:]


def _prep(x, wlt, w1at):
    return pl.pallas_call(
        _prep_body,
        grid=(N // M_BLK,),
        in_specs=[
            pl.BlockSpec((M_BLK, D), lambda i: (i, 0)),
            pl.BlockSpec((D, D), lambda i: (0, 0)),
            pl.BlockSpec((D, D), lambda i: (0, 0)),
        ],
        out_specs=[
            pl.BlockSpec((M_BLK, D), lambda i: (i, 0)),
            pl.BlockSpec((2, M_BLK, H), lambda i: (0, i, 0)),
        ],
        out_shape=[
            jax.ShapeDtypeStruct((N, D), jnp.float32),
            jax.ShapeDtypeStruct((2, N, H), jnp.float32),
        ],
    )(x, wlt, w1at)


def _edge_body(ea_ref, w1bt_ref, b1_ref, ew_ref):
    ew = (jnp.dot(ea_ref[...], w1bt_ref[...],
                  preferred_element_type=jnp.float32)
          + b1_ref[...]).astype(jnp.bfloat16)
    ew_ref[0] = ew[:, :H]
    ew_ref[1] = ew[:, H:]


def _edge(edge_attr, w1bt, b1):
    return pl.pallas_call(
        _edge_body,
        grid=(EP // E_BLK,),
        in_specs=[
            pl.BlockSpec((E_BLK, DE), lambda i: (i, 0)),
            pl.BlockSpec((DE, D), lambda i: (0, 0)),
            pl.BlockSpec((1, D), lambda i: (0, 0)),
        ],
        out_specs=[pl.BlockSpec((2, E_BLK, H), lambda i: (0, i, 0))],
        out_shape=[jax.ShapeDtypeStruct((2, EP, H), jnp.bfloat16)],
    )(edge_attr, w1bt, b1)[0]


def _out_body(ns_ref, h_ref, w2t_ref, b2_ref, o_ref):
    acc = jnp.dot(ns_ref[0], w2t_ref[0:H, :], preferred_element_type=jnp.float32)
    acc = acc + jnp.dot(ns_ref[1], w2t_ref[H:2 * H, :],
                        preferred_element_type=jnp.float32)
    acc = acc + jnp.dot(h_ref[...], w2t_ref[2 * H:, :],
                        preferred_element_type=jnp.float32)
    o_ref[...] = jnp.maximum(acc + b2_ref[...], 0.0)


def _out(ns_s, h, w2t, b2):
    return pl.pallas_call(
        _out_body,
        grid=(N // M_BLK,),
        in_specs=[
            pl.BlockSpec((2, M_BLK, H), lambda i: (0, i, 0)),
            pl.BlockSpec((M_BLK, D), lambda i: (i, 0)),
            pl.BlockSpec((2 * D, D), lambda i: (0, 0)),
            pl.BlockSpec((1, D), lambda i: (0, 0)),
        ],
        out_specs=pl.BlockSpec((M_BLK, D), lambda i: (i, 0)),
        out_shape=jax.ShapeDtypeStruct((N, D), jnp.float32),
    )(ns_s, h, w2t, b2)


@functools.cache
def _get_sc_kernel():
    mesh = plsc.VectorSubcoreMesh(core_axis_name="c", subcore_axis_name="s")

    @functools.partial(
        pl.kernel,
        mesh=mesh,
        out_type=jax.ShapeDtypeStruct((2 * NP, H), jnp.float32),
        scratch_types=[
            pltpu.VMEM((CH,), jnp.int32),
            pltpu.VMEM((CH,), jnp.int32),
            pltpu.VMEM((CH,), jnp.int32),
            pltpu.VMEM((CH,), jnp.int32),
            pltpu.VMEM((CH,), jnp.int32),
            pltpu.VMEM((CH,), jnp.int32),
            pltpu.VMEM((CH,), jnp.int32),
            pltpu.VMEM((CH,), jnp.int32),
            pltpu.VMEM((CH, H), jnp.float32),
            pltpu.VMEM((CH, H), jnp.float32),
            pltpu.VMEM((CH, H), jnp.float32),
            pltpu.VMEM((CH, H // 2), jnp.int32),
            pltpu.VMEM((CH, H // 2), jnp.int32),
            pltpu.VMEM_SHARED((NP, H), jnp.float32),
            pltpu.SemaphoreType.DMA,
            pltpu.SemaphoreType.DMA,
            pltpu.SemaphoreType.DMA,
            pltpu.SemaphoreType.DMA,
            pltpu.SemaphoreType.DMA,
            pltpu.SemaphoreType.DMA,
            pltpu.SemaphoreType.DMA,
            pltpu.SemaphoreType.DMA,
            pltpu.SemaphoreType.DMA,
            pltpu.SemaphoreType.DMA,
            pltpu.SemaphoreType.DMA,
            pltpu.SemaphoreType.DMA,
        ],
    )
    def _sc_edge_agg(hw_hbm, ew_hbm, src2_hbm, dst_hbm, zeros_hbm, out_hbm,
                     s0, s1, s2, s3, d0, d1, d2, d3, g0, g1, g2, e0, e1,
                     accum, si0, si1, si2, si3, sg0, sg1, sg2, se0, se1,
                     ss0, ss1, ss2):
        _sc_body(hw_hbm, ew_hbm, src2_hbm, dst_hbm, zeros_hbm, out_hbm,
                 s0, s1, s2, s3, d0, d1, d2, d3, g0, g1, g2, e0, e1,
                 accum, si0, si1, si2, si3, sg0, sg1, sg2, se0, se1,
                 ss0, ss1, ss2)

    return _sc_edge_agg


def _sc_body(hw_hbm, ew_hbm, src2_hbm, dst_hbm, zeros_hbm, out_hbm,
             s0, s1, s2, s3, d0, d1, d2, d3, g0, g1, g2, e0, e1,
             accum, si0, si1, si2, si3, sg0, sg1, sg2, se0, se1,
             ss0, ss1, ss2):
    c = lax.axis_index("c")
    s = lax.axis_index("s")
    ebase2 = c * EP + s * EPT

    pltpu.sync_copy(zeros_hbm.at[pl.ds(s * RPT, RPT)],
                    accum.at[pl.ds(s * RPT, RPT)])
    plsc.subcore_barrier()

    sidxs = (s0, s1, s2, s3)
    didxs = (d0, d1, d2, d3)
    gbufs = (g0, g1, g2)
    ebufs = (e0, e1)
    isem = (si0, si1, si2, si3)
    gsem = (sg0, sg1, sg2)
    esem = (se0, se1)
    ssem = (ss0, ss1, ss2)

    def start_idx(i4, k):
        off = k * CH
        pltpu.async_copy(src2_hbm.at[pl.ds(ebase2 + off, CH)],
                         sidxs[i4], isem[i4])
        pltpu.async_copy(dst_hbm.at[pl.ds(s * EPT + off, CH)],
                         didxs[i4], isem[i4])

    def wait_idx(i4, k):
        off = k * CH
        pltpu.make_async_copy(src2_hbm.at[pl.ds(ebase2 + off, CH)],
                              sidxs[i4], isem[i4]).wait()
        pltpu.make_async_copy(dst_hbm.at[pl.ds(s * EPT + off, CH)],
                              didxs[i4], isem[i4]).wait()

    def start_fetch(i4, i3, i2, k):
        pltpu.async_copy(hw_hbm.at[sidxs[i4]], gbufs[i3], gsem[i3])
        pltpu.async_copy(ew_hbm.at[pl.ds(ebase2 + k * CH, CH)],
                         ebufs[i2], esem[i2])

    def wait_scatter(i4, i3):
        pltpu.make_async_copy(gbufs[i3], accum.at[didxs[i4]],
                              ssem[i3]).wait()

    def process(k, m):
        i4, i3, i2 = m % 4, m % 3, m % 2
        p4, p3, p2 = (m + 1) % 4, (m + 1) % 3, (m + 1) % 2

        @pl.when(k >= 2)
        def _():
            wait_scatter((m - 2) % 4, (m - 2) % 3)

        @pl.when(k + 1 < N_CH)
        def _():
            wait_idx(p4, k + 1)
            start_fetch(p4, p3, p2, k + 1)

        @pl.when(k + 2 < N_CH)
        def _():
            start_idx((m + 2) % 4, k + 2)
        g, eb = gbufs[i3], ebufs[i2]
        pltpu.make_async_copy(hw_hbm.at[sidxs[i4]], g, gsem[i3]).wait()
        pltpu.make_async_copy(ew_hbm.at[pl.ds(ebase2 + k * CH, CH)],
                              eb, esem[i2]).wait()

        def row(r, rc):
            for j in range(H // 32):
                w = eb[r, pl.ds(j * 16, 16)]
                lo = lax.bitcast_convert_type(
                    lax.shift_left(w, 16), jnp.float32)
                hi = lax.bitcast_convert_type(
                    jnp.bitwise_and(w, jnp.int32(-65536)), jnp.float32)
                sla = pl.ds(j * 32, 16)
                slb = pl.ds(j * 32 + 16, 16)
                g[r, sla] = jnp.maximum(g[r, sla] + lo, 0.0)
                g[r, slb] = jnp.maximum(g[r, slb] + hi, 0.0)
            return rc
        lax.fori_loop(0, CH, row, 0)
        pltpu.async_copy(g, accum.at[didxs[i4]], ssem[i3], add=True)

    pltpu.sync_copy(src2_hbm.at[pl.ds(ebase2, CH)], s0)
    pltpu.sync_copy(dst_hbm.at[pl.ds(s * EPT, CH)], d0)
    start_fetch(0, 0, 0, 0)
    start_idx(1, 1)

    def chunk(k, carry):
        for m in range(12):
            @pl.when(k % 12 == m)
            def _(m=m):
                process(k, m)
        return carry

    lax.fori_loop(0, N_CH, chunk, 0)
    wait_scatter((N_CH - 2) % 4, (N_CH - 2) % 3)
    wait_scatter((N_CH - 1) % 4, (N_CH - 1) % 3)
    plsc.subcore_barrier()
    pltpu.sync_copy(accum.at[pl.ds(s * RPT, RPT)],
                    out_hbm.at[pl.ds(c * NP + s * RPT, RPT)])


def kernel(x, edge_index, edge_attr, W_lin, W1, b1, W2, b2):
    src = edge_index[0].astype(jnp.int32)
    dst = edge_index[1].astype(jnp.int32)
    srcp = jnp.concatenate([src, jnp.zeros((EP - E,), jnp.int32)])
    dstp = jnp.concatenate([dst, jnp.full((EP - E,), N, jnp.int32)])
    src2 = jnp.concatenate([srcp, srcp + N])
    wlt = W_lin.T
    w1at = W1[:, :D].T
    w1bt = W1[:, D:].T
    w1bt = w1bt[:, _EW_PERM]
    w2t = W2.T
    h, hw_s = _prep(x, wlt, w1at)
    ea_p = jnp.concatenate(
        [edge_attr, jnp.zeros((EP - E, DE), jnp.float32)])
    ew_s = _edge(ea_p, w1bt, b1[_EW_PERM].reshape(1, D))
    hw_flat = hw_s.reshape(2 * N, H)
    ew_flat = jax.lax.bitcast_convert_type(
        ew_s.reshape(2, EP, H // 2, 2), jnp.int32).reshape(2 * EP, H // 2)
    zeros = jnp.zeros((NP, H), jnp.float32)
    ns_flat = _get_sc_kernel()(hw_flat, ew_flat, src2, dstp, zeros)
    ns_s = ns_flat.reshape(2, NP, H)
    return _out(ns_s, h, w2t, b2.reshape(1, D))


# speedup vs baseline: 1.7625x; 1.7625x over previous
import functools

import jax
import numpy as np
import jax.numpy as jnp
from jax import lax
from jax.experimental import pallas as pl
from jax.experimental.pallas import tpu as pltpu
from jax.experimental.pallas import tpu_sc as plsc

N = 10000
E = 160000
D = 256
DE = 16
H = 128
M_BLK = 1000
E_BLK = 2048
CH = 64
N_SUB = 16
EP = 163840
EPT = EP // N_SUB
N_CH = EPT // CH
NP = 10240
RPT = NP // N_SUB


def _make_ew_perms():
    pa, pb = [], []
    for half in range(2):
        base = half * H
        for w in range(H // 2):
            pa.append(base + 32 * (w // 16) + w % 16)
            pb.append(base + 32 * (w // 16) + 16 + w % 16)
    return np.array(pa, dtype=np.int32), np.array(pb, dtype=np.int32)


_EW_PERM_A, _EW_PERM_B = _make_ew_perms()


def _prep_body(x_ref, wlt_ref, w1at_ref, h_ref, hw_ref):
    h = jnp.maximum(
        jnp.dot(x_ref[...], wlt_ref[...], preferred_element_type=jnp.float32), 0.0)
    h_ref[...] = h
    hw = jnp.dot(h, w1at_ref[...], preferred_element_type=jnp.float32)
    hw_ref[0] = hw[:, :H]
    hw_ref[1] = hw[:, H:]


def _prep(x, wlt, w1at):
    return pl.pallas_call(
        _prep_body,
        grid=(N // M_BLK,),
        in_specs=[
            pl.BlockSpec((M_BLK, D), lambda i: (i, 0)),
            pl.BlockSpec((D, D), lambda i: (0, 0)),
            pl.BlockSpec((D, D), lambda i: (0, 0)),
        ],
        out_specs=[
            pl.BlockSpec((M_BLK, D), lambda i: (i, 0)),
            pl.BlockSpec((2, M_BLK, H), lambda i: (0, i, 0)),
        ],
        out_shape=[
            jax.ShapeDtypeStruct((N, D), jnp.float32),
            jax.ShapeDtypeStruct((2, N, H), jnp.float32),
        ],
    )(x, wlt, w1at)


def _edge_body(ea_ref, wa_ref, wb_ref, ba_ref, bb_ref, ew_ref):
    a = (jnp.dot(ea_ref[...], wa_ref[...], preferred_element_type=jnp.float32)
         + ba_ref[...]).astype(jnp.bfloat16).astype(jnp.float32)
    b = (jnp.dot(ea_ref[...], wb_ref[...], preferred_element_type=jnp.float32)
         + bb_ref[...]).astype(jnp.bfloat16).astype(jnp.float32)
    abits = lax.bitcast_convert_type(a, jnp.int32)
    bbits = lax.bitcast_convert_type(b, jnp.int32)
    w = jnp.bitwise_or(lax.shift_right_logical(abits, 16),
                       jnp.bitwise_and(bbits, jnp.int32(-65536)))
    ew_ref[0] = w[:, :H // 2]
    ew_ref[1] = w[:, H // 2:]


def _edge(edge_attr, wa, wb, ba, bb):
    return pl.pallas_call(
        _edge_body,
        grid=(EP // E_BLK,),
        in_specs=[
            pl.BlockSpec((E_BLK, DE), lambda i: (i, 0)),
            pl.BlockSpec((DE, D // 2), lambda i: (0, 0)),
            pl.BlockSpec((DE, D // 2), lambda i: (0, 0)),
            pl.BlockSpec((1, D // 2), lambda i: (0, 0)),
            pl.BlockSpec((1, D // 2), lambda i: (0, 0)),
        ],
        out_specs=[pl.BlockSpec((2, E_BLK, H // 2), lambda i: (0, i, 0))],
        out_shape=[jax.ShapeDtypeStruct((2, EP, H // 2), jnp.int32)],
    )(edge_attr, wa, wb, ba, bb)[0]


def _out_body(ns_ref, h_ref, w2t_ref, b2_ref, o_ref):
    acc = jnp.dot(ns_ref[0], w2t_ref[0:H, :], preferred_element_type=jnp.float32)
    acc = acc + jnp.dot(ns_ref[1], w2t_ref[H:2 * H, :],
                        preferred_element_type=jnp.float32)
    acc = acc + jnp.dot(h_ref[...], w2t_ref[2 * H:, :],
                        preferred_element_type=jnp.float32)
    o_ref[...] = jnp.maximum(acc + b2_ref[...], 0.0)


def _out(ns_s, h, w2t, b2):
    return pl.pallas_call(
        _out_body,
        grid=(N // M_BLK,),
        in_specs=[
            pl.BlockSpec((2, M_BLK, H), lambda i: (0, i, 0)),
            pl.BlockSpec((M_BLK, D), lambda i: (i, 0)),
            pl.BlockSpec((2 * D, D), lambda i: (0, 0)),
            pl.BlockSpec((1, D), lambda i: (0, 0)),
        ],
        out_specs=pl.BlockSpec((M_BLK, D), lambda i: (i, 0)),
        out_shape=jax.ShapeDtypeStruct((N, D), jnp.float32),
    )(ns_s, h, w2t, b2)


@functools.cache
def _get_sc_kernel():
    mesh = plsc.VectorSubcoreMesh(core_axis_name="c", subcore_axis_name="s")

    @functools.partial(
        pl.kernel,
        mesh=mesh,
        out_type=jax.ShapeDtypeStruct((2 * NP, H), jnp.float32),
        scratch_types=[
            pltpu.VMEM((CH,), jnp.int32),
            pltpu.VMEM((CH,), jnp.int32),
            pltpu.VMEM((CH,), jnp.int32),
            pltpu.VMEM((CH,), jnp.int32),
            pltpu.VMEM((CH,), jnp.int32),
            pltpu.VMEM((CH,), jnp.int32),
            pltpu.VMEM((CH,), jnp.int32),
            pltpu.VMEM((CH,), jnp.int32),
            pltpu.VMEM((CH, H), jnp.float32),
            pltpu.VMEM((CH, H), jnp.float32),
            pltpu.VMEM((CH, H), jnp.float32),
            pltpu.VMEM((CH, H // 2), jnp.int32),
            pltpu.VMEM((CH, H // 2), jnp.int32),
            pltpu.VMEM_SHARED((NP, H), jnp.float32),
            pltpu.SemaphoreType.DMA,
            pltpu.SemaphoreType.DMA,
            pltpu.SemaphoreType.DMA,
            pltpu.SemaphoreType.DMA,
            pltpu.SemaphoreType.DMA,
            pltpu.SemaphoreType.DMA,
            pltpu.SemaphoreType.DMA,
            pltpu.SemaphoreType.DMA,
            pltpu.SemaphoreType.DMA,
            pltpu.SemaphoreType.DMA,
            pltpu.SemaphoreType.DMA,
            pltpu.SemaphoreType.DMA,
        ],
    )
    def _sc_edge_agg(hw_hbm, ew_hbm, src2_hbm, dst_hbm, zeros_hbm, out_hbm,
                     s0, s1, s2, s3, d0, d1, d2, d3, g0, g1, g2, e0, e1,
                     accum, si0, si1, si2, si3, sg0, sg1, sg2, se0, se1,
                     ss0, ss1, ss2):
        _sc_body(hw_hbm, ew_hbm, src2_hbm, dst_hbm, zeros_hbm, out_hbm,
                 s0, s1, s2, s3, d0, d1, d2, d3, g0, g1, g2, e0, e1,
                 accum, si0, si1, si2, si3, sg0, sg1, sg2, se0, se1,
                 ss0, ss1, ss2)

    return _sc_edge_agg


def _sc_body(hw_hbm, ew_hbm, src2_hbm, dst_hbm, zeros_hbm, out_hbm,
             s0, s1, s2, s3, d0, d1, d2, d3, g0, g1, g2, e0, e1,
             accum, si0, si1, si2, si3, sg0, sg1, sg2, se0, se1,
             ss0, ss1, ss2):
    c = lax.axis_index("c")
    s = lax.axis_index("s")
    ebase2 = c * EP + s * EPT

    pltpu.sync_copy(zeros_hbm.at[pl.ds(s * RPT, RPT)],
                    accum.at[pl.ds(s * RPT, RPT)])
    plsc.subcore_barrier()

    sidxs = (s0, s1, s2, s3)
    didxs = (d0, d1, d2, d3)
    gbufs = (g0, g1, g2)
    ebufs = (e0, e1)
    isem = (si0, si1, si2, si3)
    gsem = (sg0, sg1, sg2)
    esem = (se0, se1)
    ssem = (ss0, ss1, ss2)

    def start_idx(i4, k):
        off = k * CH
        pltpu.async_copy(src2_hbm.at[pl.ds(ebase2 + off, CH)],
                         sidxs[i4], isem[i4])
        pltpu.async_copy(dst_hbm.at[pl.ds(s * EPT + off, CH)],
                         didxs[i4], isem[i4])

    def wait_idx(i4, k):
        off = k * CH
        pltpu.make_async_copy(src2_hbm.at[pl.ds(ebase2 + off, CH)],
                              sidxs[i4], isem[i4]).wait()
        pltpu.make_async_copy(dst_hbm.at[pl.ds(s * EPT + off, CH)],
                              didxs[i4], isem[i4]).wait()

    def start_fetch(i4, i3, i2, k):
        pltpu.async_copy(hw_hbm.at[sidxs[i4]], gbufs[i3], gsem[i3])
        pltpu.async_copy(ew_hbm.at[pl.ds(ebase2 + k * CH, CH)],
                         ebufs[i2], esem[i2])

    def wait_scatter(i4, i3):
        pltpu.make_async_copy(gbufs[i3], accum.at[didxs[i4]],
                              ssem[i3]).wait()

    def process(k, m):
        i4, i3, i2 = m % 4, m % 3, m % 2
        p4, p3, p2 = (m + 1) % 4, (m + 1) % 3, (m + 1) % 2

        @pl.when(k >= 2)
        def _():
            wait_scatter((m - 2) % 4, (m - 2) % 3)

        @pl.when(k + 1 < N_CH)
        def _():
            wait_idx(p4, k + 1)
            start_fetch(p4, p3, p2, k + 1)

        @pl.when(k + 2 < N_CH)
        def _():
            start_idx((m + 2) % 4, k + 2)
        g, eb = gbufs[i3], ebufs[i2]
        pltpu.make_async_copy(hw_hbm.at[sidxs[i4]], g, gsem[i3]).wait()
        pltpu.make_async_copy(ew_hbm.at[pl.ds(ebase2 + k * CH, CH)],
                              eb, esem[i2]).wait()

        def row(r, rc):
            for j in range(H // 32):
                w = eb[r, pl.ds(j * 16, 16)]
                lo = lax.bitcast_convert_type(
                    lax.shift_left(w, 16), jnp.float32)
                hi = lax.bitcast_convert_type(
                    jnp.bitwise_and(w, jnp.int32(-65536)), jnp.float32)
                sla = pl.ds(j * 32, 16)
                slb = pl.ds(j * 32 + 16, 16)
                g[r, sla] = jnp.maximum(g[r, sla] + lo, 0.0)
                g[r, slb] = jnp.maximum(g[r, slb] + hi, 0.0)
            return rc
        lax.fori_loop(0, CH, row, 0)
        pltpu.async_copy(g, accum.at[didxs[i4]], ssem[i3], add=True)

    pltpu.sync_copy(src2_hbm.at[pl.ds(ebase2, CH)], s0)
    pltpu.sync_copy(dst_hbm.at[pl.ds(s * EPT, CH)], d0)
    start_fetch(0, 0, 0, 0)
    start_idx(1, 1)

    def chunk(k, carry):
        for m in range(12):
            @pl.when(k % 12 == m)
            def _(m=m):
                process(k, m)
        return carry

    lax.fori_loop(0, N_CH, chunk, 0)
    wait_scatter((N_CH - 2) % 4, (N_CH - 2) % 3)
    wait_scatter((N_CH - 1) % 4, (N_CH - 1) % 3)
    plsc.subcore_barrier()
    pltpu.sync_copy(accum.at[pl.ds(s * RPT, RPT)],
                    out_hbm.at[pl.ds(c * NP + s * RPT, RPT)])


def kernel(x, edge_index, edge_attr, W_lin, W1, b1, W2, b2):
    src = edge_index[0].astype(jnp.int32)
    dst = edge_index[1].astype(jnp.int32)
    srcp = jnp.concatenate([src, jnp.zeros((EP - E,), jnp.int32)])
    dstp = jnp.concatenate([dst, jnp.full((EP - E,), N, jnp.int32)])
    src2 = jnp.concatenate([srcp, srcp + N])
    wlt = W_lin.T
    w1at = W1[:, :D].T
    w1bt = W1[:, D:].T
    wa = w1bt[:, _EW_PERM_A]
    wb = w1bt[:, _EW_PERM_B]
    ba = b1[_EW_PERM_A].reshape(1, D // 2)
    bb = b1[_EW_PERM_B].reshape(1, D // 2)
    w2t = W2.T
    h, hw_s = _prep(x, wlt, w1at)
    ea_p = jnp.concatenate(
        [edge_attr, jnp.zeros((EP - E, DE), jnp.float32)])
    ew_s = _edge(ea_p, wa, wb, ba, bb)
    hw_flat = hw_s.reshape(2 * N, H)
    ew_flat = ew_s.reshape(2 * EP, H // 2)
    zeros = jnp.zeros((NP, H), jnp.float32)
    ns_flat = _get_sc_kernel()(hw_flat, ew_flat, src2, dstp, zeros)
    ns_s = ns_flat.reshape(2, NP, H)
    return _out(ns_s, h, w2t, b2.reshape(1, D))


# speedup vs baseline: 1.7643x; 1.0010x over previous
import functools

import jax
import jax.numpy as jnp
from jax import lax
from jax.experimental import pallas as pl
from jax.experimental.pallas import tpu as pltpu
from jax.experimental.pallas import tpu_sc as plsc

N = 10000
E = 160000
D = 256
DE = 16
H = 128
M_BLK = 1000
E_BLK = 2048
CH = 64
N_SUB = 16
EP = 163840
EPT = EP // N_SUB
N_CH = EPT // CH
NP = 10240
RPT = NP // N_SUB


def _prep_body(x_ref, wlt_ref, w1at_ref, h_ref, hw_ref):
    h = jnp.maximum(
        jnp.dot(x_ref[...], wlt_ref[...], preferred_element_type=jnp.float32), 0.0)
    h_ref[...] = h
    hw = jnp.dot(h, w1at_ref[...], preferred_element_type=jnp.float32)
    hw_ref[0] = hw[:, :H]
    hw_ref[1] = hw[:, H:]


def _prep(x, wlt, w1at):
    return pl.pallas_call(
        _prep_body,
        grid=(N // M_BLK,),
        in_specs=[
            pl.BlockSpec((M_BLK, D), lambda i: (i, 0)),
            pl.BlockSpec((D, D), lambda i: (0, 0)),
            pl.BlockSpec((D, D), lambda i: (0, 0)),
        ],
        out_specs=[
            pl.BlockSpec((M_BLK, D), lambda i: (i, 0)),
            pl.BlockSpec((2, M_BLK, H), lambda i: (0, i, 0)),
        ],
        out_shape=[
            jax.ShapeDtypeStruct((N, D), jnp.float32),
            jax.ShapeDtypeStruct((2, N, H), jnp.float32),
        ],
    )(x, wlt, w1at)


def _edge_body(ea_ref, w1bt_ref, b1_ref, ew_ref):
    ew = jnp.dot(ea_ref[...], w1bt_ref[...],
                 preferred_element_type=jnp.float32) + b1_ref[...]
    ew_ref[0] = ew[:, :H]
    ew_ref[1] = ew[:, H:]


def _edge(edge_attr, w1bt, b1):
    rows = edge_attr.shape[0]
    return pl.pallas_call(
        _edge_body,
        grid=(rows // E_BLK,),
        in_specs=[
            pl.BlockSpec((E_BLK, DE), lambda i: (i, 0)),
            pl.BlockSpec((DE, D), lambda i: (0, 0)),
            pl.BlockSpec((1, D), lambda i: (0, 0)),
        ],
        out_specs=[pl.BlockSpec((2, E_BLK, H), lambda i: (0, i, 0))],
        out_shape=[jax.ShapeDtypeStruct((2, rows, H), jnp.float32)],
    )(edge_attr, w1bt, b1)[0]


def _out_body(ns_ref, ns2_ref, h_ref, w2t_ref, b2_ref, o_ref):
    acc = jnp.dot(ns_ref[0] + ns2_ref[0], w2t_ref[0:H, :],
                  preferred_element_type=jnp.float32)
    acc = acc + jnp.dot(ns_ref[1] + ns2_ref[1], w2t_ref[H:2 * H, :],
                        preferred_element_type=jnp.float32)
    acc = acc + jnp.dot(h_ref[...], w2t_ref[2 * H:, :],
                        preferred_element_type=jnp.float32)
    o_ref[...] = jnp.maximum(acc + b2_ref[...], 0.0)


def _out(ns_s, ns2_s, h, w2t, b2):
    return pl.pallas_call(
        _out_body,
        grid=(N // M_BLK,),
        in_specs=[
            pl.BlockSpec((2, M_BLK, H), lambda i: (0, i, 0)),
            pl.BlockSpec((2, M_BLK, H), lambda i: (0, i, 0)),
            pl.BlockSpec((M_BLK, D), lambda i: (i, 0)),
            pl.BlockSpec((2 * D, D), lambda i: (0, 0)),
            pl.BlockSpec((1, D), lambda i: (0, 0)),
        ],
        out_specs=pl.BlockSpec((M_BLK, D), lambda i: (i, 0)),
        out_shape=jax.ShapeDtypeStruct((N, D), jnp.float32),
    )(ns_s, ns2_s, h, w2t, b2)


@functools.cache
def _get_sc_kernel(ept):
    mesh = plsc.VectorSubcoreMesh(core_axis_name="c", subcore_axis_name="s")

    @functools.partial(
        pl.kernel,
        mesh=mesh,
        out_type=jax.ShapeDtypeStruct((2 * NP, H), jnp.float32),
        scratch_types=[
            pltpu.VMEM((CH,), jnp.int32),
            pltpu.VMEM((CH,), jnp.int32),
            pltpu.VMEM((CH,), jnp.int32),
            pltpu.VMEM((CH,), jnp.int32),
            pltpu.VMEM((CH,), jnp.int32),
            pltpu.VMEM((CH,), jnp.int32),
            pltpu.VMEM((CH,), jnp.int32),
            pltpu.VMEM((CH,), jnp.int32),
            pltpu.VMEM((CH, H), jnp.float32),
            pltpu.VMEM((CH, H), jnp.float32),
            pltpu.VMEM((CH, H), jnp.float32),
            pltpu.VMEM((CH, H), jnp.float32),
            pltpu.VMEM_SHARED((NP, H), jnp.float32),
            pltpu.SemaphoreType.DMA,
            pltpu.SemaphoreType.DMA,
            pltpu.SemaphoreType.DMA,
            pltpu.SemaphoreType.DMA,
            pltpu.SemaphoreType.DMA,
            pltpu.SemaphoreType.DMA,
            pltpu.SemaphoreType.DMA,
            pltpu.SemaphoreType.DMA,
        ],
    )
    def _sc_edge_agg(hw_hbm, ew_hbm, src2_hbm, dst_hbm, zeros_hbm, out_hbm,
                     s0, s1, s2, s3, d0, d1, d2, d3,
                     g0, g1, e0, e1, accum,
                     si0, si1, si2, si3, sg0, sg1, se0, se1):
        _sc_body(ept, hw_hbm, ew_hbm, src2_hbm, dst_hbm, zeros_hbm, out_hbm,
                 s0, s1, s2, s3, d0, d1, d2, d3,
                 g0, g1, e0, e1, accum,
                 si0, si1, si2, si3, sg0, sg1, se0, se1)

    return _sc_edge_agg


def _sc_body(ept, hw_hbm, ew_hbm, src2_hbm, dst_hbm, zeros_hbm, out_hbm,
             s0, s1, s2, s3, d0, d1, d2, d3,
             g0, g1, e0, e1, accum,
             si0, si1, si2, si3, sg0, sg1, se0, se1):
    n_ch = ept // CH
    c = lax.axis_index("c")
    s = lax.axis_index("s")
    ebase2 = c * (N_SUB * ept) + s * ept
    pltpu.sync_copy(zeros_hbm.at[pl.ds(s * RPT, RPT)],
                    accum.at[pl.ds(s * RPT, RPT)])
    plsc.subcore_barrier()

    isets = ((s0, d0, si0), (s1, d1, si1), (s2, d2, si2), (s3, d3, si3))
    dsets = ((g0, e0, sg0, se0), (g1, e1, sg1, se1))

    def start_idx(iset, k):
        off = k * CH
        pltpu.async_copy(src2_hbm.at[pl.ds(ebase2 + off, CH)], iset[0], iset[2])
        pltpu.async_copy(dst_hbm.at[pl.ds(s * ept + off, CH)], iset[1], iset[2])

    def wait_idx(iset, k):
        off = k * CH
        pltpu.make_async_copy(src2_hbm.at[pl.ds(ebase2 + off, CH)],
                              iset[0], iset[2]).wait()
        pltpu.make_async_copy(dst_hbm.at[pl.ds(s * ept + off, CH)],
                              iset[1], iset[2]).wait()

    def start_fetch(iset, dset, k):
        pltpu.async_copy(hw_hbm.at[iset[0]], dset[0], dset[2])
        pltpu.async_copy(ew_hbm.at[pl.ds(ebase2 + k * CH, CH)], dset[1], dset[3])

    def process(ia, ib, ic, da, db, k):
        @pl.when(k + 1 < n_ch)
        def _():
            wait_idx(ib, k + 1)
            start_fetch(ib, db, k + 1)

        @pl.when(k + 2 < n_ch)
        def _():
            start_idx(ic, k + 2)
        g, eb = da[0], da[1]
        pltpu.make_async_copy(hw_hbm.at[ia[0]], g, da[2]).wait()
        pltpu.make_async_copy(ew_hbm.at[pl.ds(ebase2 + k * CH, CH)],
                              eb, da[3]).wait()

        def row(r, rc):
            for j in range(H // 16):
                sl = pl.ds(j * 16, 16)
                g[r, sl] = jnp.maximum(g[r, sl] + eb[r, sl], 0.0)
            return rc
        lax.fori_loop(0, CH, row, 0)
        pltpu.sync_copy(g, accum.at[ia[1]], add=True)

    pltpu.sync_copy(src2_hbm.at[pl.ds(ebase2, CH)], s0)
    pltpu.sync_copy(dst_hbm.at[pl.ds(s * ept, CH)], d0)
    start_fetch(isets[0], dsets[0], 0)
    start_idx(isets[1], 1)

    def chunk(k, carry):
        for m in range(4):
            @pl.when(k % 4 == m)
            def _(m=m):
                process(isets[m], isets[(m + 1) % 4], isets[(m + 2) % 4],
                        dsets[m % 2], dsets[(m + 1) % 2], k)
        return carry

    lax.fori_loop(0, n_ch, chunk, 0)
    plsc.subcore_barrier()
    pltpu.sync_copy(accum.at[pl.ds(s * RPT, RPT)],
                    out_hbm.at[pl.ds(c * NP + s * RPT, RPT)])


def kernel(x, edge_index, edge_attr, W_lin, W1, b1, W2, b2):
    src = edge_index[0].astype(jnp.int32)
    dst = edge_index[1].astype(jnp.int32)
    srcp = jnp.concatenate([src, jnp.zeros((EP - E,), jnp.int32)])
    dstp = jnp.concatenate([dst, jnp.full((EP - E,), N, jnp.int32)])
    eph = EP // 2
    srcp_a, srcp_b = srcp[:eph], srcp[eph:]
    dstp_a, dstp_b = dstp[:eph], dstp[eph:]
    src2_a = jnp.concatenate([srcp_a, srcp_a + N])
    src2_b = jnp.concatenate([srcp_b, srcp_b + N])
    wlt = W_lin.T
    w1at = W1[:, :D].T
    w1bt = W1[:, D:].T
    w2t = W2.T
    h, hw_s = _prep(x, wlt, w1at)
    ea_p = jnp.concatenate(
        [edge_attr, jnp.zeros((EP - E, DE), jnp.float32)])
    hw_flat = hw_s.reshape(2 * N, H)
    zeros = jnp.zeros((NP, H), jnp.float32)
    sck = _get_sc_kernel(eph // N_SUB)
    ew_a = _edge(ea_p[:eph], w1bt, b1.reshape(1, D)).reshape(2 * eph, H)
    ns_a = sck(hw_flat, ew_a, src2_a, dstp_a, zeros)
    ew_b = _edge(ea_p[eph:], w1bt, b1.reshape(1, D)).reshape(2 * eph, H)
    ns_b = sck(hw_flat, ew_b, src2_b, dstp_b, zeros)
    return _out(ns_a.reshape(2, NP, H), ns_b.reshape(2, NP, H),
                h, w2t, b2.reshape(1, D))


# speedup vs baseline: 1.8242x; 1.0340x over previous
import functools

import jax
import jax.numpy as jnp
from jax import lax
from jax.experimental import pallas as pl
from jax.experimental.pallas import tpu as pltpu
from jax.experimental.pallas import tpu_sc as plsc

N = 10000
E = 160000
D = 256
DE = 16
H = 128
M_BLK = 1000
E_BLK = 2048
CH = 64
N_SUB = 16
EP = 163840
EPT = EP // N_SUB
N_CH = EPT // CH
NP = 10240
RPT = NP // N_SUB


def _prep_body(x_ref, wlt_ref, w1at_ref, h_ref, hw_ref):
    h = jnp.maximum(
        jnp.dot(x_ref[...], wlt_ref[...], preferred_element_type=jnp.float32), 0.0)
    h_ref[...] = h
    hw = jnp.dot(h, w1at_ref[...], preferred_element_type=jnp.float32)
    hw_ref[0] = hw[:, :H]
    hw_ref[1] = hw[:, H:]


def _prep(x, wlt, w1at):
    return pl.pallas_call(
        _prep_body,
        grid=(N // M_BLK,),
        in_specs=[
            pl.BlockSpec((M_BLK, D), lambda i: (i, 0)),
            pl.BlockSpec((D, D), lambda i: (0, 0)),
            pl.BlockSpec((D, D), lambda i: (0, 0)),
        ],
        out_specs=[
            pl.BlockSpec((M_BLK, D), lambda i: (i, 0)),
            pl.BlockSpec((2, M_BLK, H), lambda i: (0, i, 0)),
        ],
        out_shape=[
            jax.ShapeDtypeStruct((N, D), jnp.float32),
            jax.ShapeDtypeStruct((2, N, H), jnp.float32),
        ],
    )(x, wlt, w1at)


def _edge_body(ea_ref, w1bt_ref, b1_ref, ew_ref):
    ew = jnp.dot(ea_ref[...], w1bt_ref[...],
                 preferred_element_type=jnp.float32) + b1_ref[...]
    ew_ref[0] = ew[:, :H]
    ew_ref[1] = ew[:, H:]


def _edge(edge_attr, w1bt, b1):
    return pl.pallas_call(
        _edge_body,
        grid=(EP // E_BLK,),
        in_specs=[
            pl.BlockSpec((E_BLK, DE), lambda i: (i, 0)),
            pl.BlockSpec((DE, D), lambda i: (0, 0)),
            pl.BlockSpec((1, D), lambda i: (0, 0)),
        ],
        out_specs=[pl.BlockSpec((2, E_BLK, H), lambda i: (0, i, 0))],
        out_shape=[jax.ShapeDtypeStruct((2, EP, H), jnp.float32)],
    )(edge_attr, w1bt, b1)[0]


def _out_body(ns_ref, h_ref, w2t_ref, b2_ref, o_ref):
    acc = jnp.dot(ns_ref[0], w2t_ref[0:H, :], preferred_element_type=jnp.float32)
    acc = acc + jnp.dot(ns_ref[1], w2t_ref[H:2 * H, :],
                        preferred_element_type=jnp.float32)
    acc = acc + jnp.dot(h_ref[...], w2t_ref[2 * H:, :],
                        preferred_element_type=jnp.float32)
    o_ref[...] = jnp.maximum(acc + b2_ref[...], 0.0)


def _out(ns_s, h, w2t, b2):
    return pl.pallas_call(
        _out_body,
        grid=(N // M_BLK,),
        in_specs=[
            pl.BlockSpec((2, M_BLK, H), lambda i: (0, i, 0)),
            pl.BlockSpec((M_BLK, D), lambda i: (i, 0)),
            pl.BlockSpec((2 * D, D), lambda i: (0, 0)),
            pl.BlockSpec((1, D), lambda i: (0, 0)),
        ],
        out_specs=pl.BlockSpec((M_BLK, D), lambda i: (i, 0)),
        out_shape=jax.ShapeDtypeStruct((N, D), jnp.float32),
    )(ns_s, h, w2t, b2)


@functools.cache
def _get_sc_kernel():
    mesh = plsc.VectorSubcoreMesh(core_axis_name="c", subcore_axis_name="s")

    @functools.partial(
        pl.kernel,
        mesh=mesh,
        out_type=jax.ShapeDtypeStruct((2 * NP, H), jnp.float32),
        scratch_types=[
            pltpu.VMEM((CH,), jnp.int32),
            pltpu.VMEM((CH,), jnp.int32),
            pltpu.VMEM((CH,), jnp.int32),
            pltpu.VMEM((CH,), jnp.int32),
            pltpu.VMEM((CH,), jnp.int32),
            pltpu.VMEM((CH,), jnp.int32),
            pltpu.VMEM((CH,), jnp.int32),
            pltpu.VMEM((CH,), jnp.int32),
            pltpu.VMEM((CH, H), jnp.float32),
            pltpu.VMEM((CH, H), jnp.float32),
            pltpu.VMEM((CH, H), jnp.float32),
            pltpu.VMEM((CH, H), jnp.float32),
            pltpu.VMEM_SHARED((NP, H), jnp.float32),
            pltpu.SemaphoreType.DMA,
            pltpu.SemaphoreType.DMA,
            pltpu.SemaphoreType.DMA,
            pltpu.SemaphoreType.DMA,
            pltpu.SemaphoreType.DMA,
            pltpu.SemaphoreType.DMA,
            pltpu.SemaphoreType.DMA,
            pltpu.SemaphoreType.DMA,
        ],
    )
    def _sc_edge_agg(hw_hbm, ew_hbm, src2_hbm, dst_hbm, zeros_hbm, out_hbm,
                     s0, s1, s2, s3, d0, d1, d2, d3,
                     g0, g1, e0, e1, accum,
                     si0, si1, si2, si3, sg0, sg1, se0, se1):
        _sc_body(hw_hbm, ew_hbm, src2_hbm, dst_hbm, zeros_hbm, out_hbm,
                 s0, s1, s2, s3, d0, d1, d2, d3,
                 g0, g1, e0, e1, accum,
                 si0, si1, si2, si3, sg0, sg1, se0, se1)

    return _sc_edge_agg


def _sc_body(hw_hbm, ew_hbm, src2_hbm, dst_hbm, zeros_hbm, out_hbm,
             s0, s1, s2, s3, d0, d1, d2, d3,
             g0, g1, e0, e1, accum,
             si0, si1, si2, si3, sg0, sg1, se0, se1):
    c = lax.axis_index("c")
    s = lax.axis_index("s")
    ebase2 = c * EP + s * EPT
    pltpu.sync_copy(zeros_hbm.at[pl.ds(s * RPT, RPT)],
                    accum.at[pl.ds(s * RPT, RPT)])
    plsc.subcore_barrier()

    isets = ((s0, d0, si0), (s1, d1, si1), (s2, d2, si2), (s3, d3, si3))
    dsets = ((g0, e0, sg0, se0), (g1, e1, sg1, se1))

    def start_idx(iset, k):
        off = k * CH
        pltpu.async_copy(src2_hbm.at[pl.ds(ebase2 + off, CH)], iset[0], iset[2])
        pltpu.async_copy(dst_hbm.at[pl.ds(s * EPT + off, CH)], iset[1], iset[2])

    def wait_idx(iset, k):
        off = k * CH
        pltpu.make_async_copy(src2_hbm.at[pl.ds(ebase2 + off, CH)],
                              iset[0], iset[2]).wait()
        pltpu.make_async_copy(dst_hbm.at[pl.ds(s * EPT + off, CH)],
                              iset[1], iset[2]).wait()

    def start_fetch(iset, dset, k):
        pltpu.async_copy(hw_hbm.at[iset[0]], dset[0], dset[2])
        pltpu.async_copy(ew_hbm.at[pl.ds(ebase2 + k * CH, CH)], dset[1], dset[3])

    def process(ia, ib, ic, da, db, k):
        @pl.when(k + 1 < N_CH)
        def _():
            wait_idx(ib, k + 1)
            start_fetch(ib, db, k + 1)

        @pl.when(k + 2 < N_CH)
        def _():
            start_idx(ic, k + 2)
        g, eb = da[0], da[1]
        pltpu.make_async_copy(hw_hbm.at[ia[0]], g, da[2]).wait()
        pltpu.make_async_copy(ew_hbm.at[pl.ds(ebase2 + k * CH, CH)],
                              eb, da[3]).wait()

        def row(r, rc):
            for j in range(H // 16):
                sl = pl.ds(j * 16, 16)
                g[r, sl] = jnp.maximum(g[r, sl] + eb[r, sl], 0.0)
            return rc
        lax.fori_loop(0, CH, row, 0)
        pltpu.sync_copy(g, accum.at[ia[1]], add=True)

    pltpu.sync_copy(src2_hbm.at[pl.ds(ebase2, CH)], s0)
    pltpu.sync_copy(dst_hbm.at[pl.ds(s * EPT, CH)], d0)
    start_fetch(isets[0], dsets[0], 0)
    start_idx(isets[1], 1)

    def chunk(k, carry):
        for m in range(4):
            @pl.when(k % 4 == m)
            def _(m=m):
                process(isets[m], isets[(m + 1) % 4], isets[(m + 2) % 4],
                        dsets[m % 2], dsets[(m + 1) % 2], k)
        return carry

    lax.fori_loop(0, N_CH, chunk, 0)
    plsc.subcore_barrier()
    pltpu.sync_copy(accum.at[pl.ds(s * RPT, RPT)],
                    out_hbm.at[pl.ds(c * NP + s * RPT, RPT)])


def kernel(x, edge_index, edge_attr, W_lin, W1, b1, W2, b2):
    src = edge_index[0].astype(jnp.int32)
    dst = edge_index[1].astype(jnp.int32)
    srcp = jnp.concatenate([src, jnp.zeros((EP - E,), jnp.int32)])
    dstp = jnp.concatenate([dst, jnp.full((EP - E,), N, jnp.int32)])
    src2 = jnp.concatenate([srcp, srcp + N])
    wlt = W_lin.T
    w1at = W1[:, :D].T
    w1bt = W1[:, D:].T
    w2t = W2.T
    h, hw_s = _prep(x, wlt, w1at)
    ea_p = jnp.concatenate(
        [edge_attr, jnp.zeros((EP - E, DE), jnp.float32)])
    ew_s = _edge(ea_p, w1bt, b1.reshape(1, D))
    hw_flat = hw_s.reshape(2 * N, H)
    ew_flat = ew_s.reshape(2 * EP, H)
    zeros = jnp.zeros((NP, H), jnp.float32)
    ns_flat = _get_sc_kernel()(hw_flat, ew_flat, src2, dstp, zeros)
    ns_s = ns_flat.reshape(2, NP, H)
    return _out(ns_s, h, w2t, b2.reshape(1, D))
